# baseline (speedup 1.0000x reference)
import functools

import jax
import jax.numpy as jnp
from jax import lax
from jax.experimental import pallas as pl
from jax.experimental.pallas import tpu as pltpu
from jax.experimental.pallas import tpu_sc as plsc

NC = 2
NS = 16
L = 16
NW = NC * NS
BPW = 160
KPAD = NW * BPW
NSTREAM = BPW * 4 // 128


def _make_gather(n):
    mesh = plsc.VectorSubcoreMesh(core_axis_name="c", subcore_axis_name="s")

    @functools.partial(
        pl.kernel,
        mesh=mesh,
        out_type=jax.ShapeDtypeStruct((4 * KPAD,), jnp.float32),
        compiler_params=pltpu.CompilerParams(
            needs_layout_passes=False, use_tc_tiling_on_sc=False,
            disable_bounds_checks=True, disable_semaphore_checks=True,
            skip_device_barrier=True,
        ),
        scratch_types=[
            pltpu.VMEM((BPW,), jnp.int32),
            pltpu.VMEM((BPW * 4,), jnp.int32),
            pltpu.VMEM((BPW * 4,), jnp.float32),
            pltpu.SemaphoreType.DMA,
        ],
    )
    def gather(idx_hbm, table_hbm, out_hbm, bidx, widx, vals, sem):
        wid = lax.axis_index("s") * NC + lax.axis_index("c")
        base = wid * BPW
        pltpu.sync_copy(idx_hbm.at[pl.ds(base, BPW)], bidx)
        for g in range(BPW // L):
            b16 = bidx[pl.ds(g * L, L)]
            for c in range(4):
                widx[pl.ds(c * BPW + g * L, L)] = b16 + c * n
        cps = [
            pltpu.async_copy(
                table_hbm.at[widx.at[pl.ds(s * 128, 128)]],
                vals.at[pl.ds(s * 128, 128)], sem)
            for s in range(NSTREAM)
        ]
        for cp in cps:
            cp.wait()
        for c in range(4):
            pltpu.sync_copy(vals.at[pl.ds(c * BPW, BPW)],
                            out_hbm.at[pl.ds(c * KPAD + base, BPW)])

    return gather


def kernel(selected_indices, xyxy_boxes):
    k = selected_indices.shape[0]
    n = xyxy_boxes.shape[1]
    box_idx = jnp.pad(selected_indices[:, 2].astype(jnp.int32),
                      (0, KPAD - k))
    table_cm = xyxy_boxes[0].T.reshape(-1)
    out = _make_gather(n)(box_idx, table_cm)
    return out.reshape(4, KPAD)[:, :k].T


# speedup vs baseline: 1.0272x; 1.0272x over previous
import functools

import jax
import jax.numpy as jnp
from jax import lax
from jax.experimental import pallas as pl
from jax.experimental.pallas import tpu as pltpu
from jax.experimental.pallas import tpu_sc as plsc

NC = 1
NS = 16
L = 16
NW = NC * NS
BPW = 320
KPAD = NW * BPW
NSTREAM = BPW * 4 // 128


def _make_gather(n):
    mesh = plsc.VectorSubcoreMesh(
        core_axis_name="c", subcore_axis_name="s", num_cores=NC
    )

    @functools.partial(
        pl.kernel,
        mesh=mesh,
        out_type=jax.ShapeDtypeStruct((4 * KPAD,), jnp.float32),
        compiler_params=pltpu.CompilerParams(
            needs_layout_passes=False, use_tc_tiling_on_sc=False,
            disable_bounds_checks=True, disable_semaphore_checks=True,
            skip_device_barrier=True,
        ),
        scratch_types=[
            pltpu.VMEM((BPW,), jnp.int32),
            pltpu.VMEM((BPW * 4,), jnp.int32),
            pltpu.VMEM((BPW * 4,), jnp.float32),
            pltpu.SemaphoreType.DMA,
        ],
    )
    def gather(idx_hbm, table_hbm, out_hbm, bidx, widx, vals, sem):
        wid = lax.axis_index("s") * NC + lax.axis_index("c")
        base = wid * BPW
        pltpu.sync_copy(idx_hbm.at[pl.ds(base, BPW)], bidx)
        for g in range(BPW // L):
            b16 = bidx[pl.ds(g * L, L)]
            for c in range(4):
                widx[pl.ds(c * BPW + g * L, L)] = b16 + c * n
        cps = [
            pltpu.async_copy(
                table_hbm.at[widx.at[pl.ds(s * 128, 128)]],
                vals.at[pl.ds(s * 128, 128)], sem)
            for s in range(NSTREAM)
        ]
        for cp in cps:
            cp.wait()
        for c in range(4):
            pltpu.sync_copy(vals.at[pl.ds(c * BPW, BPW)],
                            out_hbm.at[pl.ds(c * KPAD + base, BPW)])

    return gather


def kernel(selected_indices, xyxy_boxes):
    k = selected_indices.shape[0]
    n = xyxy_boxes.shape[1]
    box_idx = jnp.pad(selected_indices[:, 2].astype(jnp.int32),
                      (0, KPAD - k))
    table_cm = xyxy_boxes[0].T.reshape(-1)
    out = _make_gather(n)(box_idx, table_cm)
    return out.reshape(4, KPAD)[:, :k].T
